# baseline (speedup 1.0000x reference)
import jax
import jax.numpy as jnp
from jax.experimental import pallas as pl
from jax.experimental.pallas import tpu as pltpu

_CHUNK = 1024
_HALF = _CHUNK // 2
_NBUF = 10


def _router_kernel(x_hbm, wl_ref, wr_ref, out_hbm, xbuf, obuf, isem, osem):
    n_chunks = x_hbm.shape[0] // _CHUNK
    wl = wl_ref[...].astype(jnp.bfloat16)
    wr = wr_ref[...].astype(jnp.bfloat16)

    def in_copy(c, slot):
        return pltpu.make_async_copy(
            x_hbm.at[pl.ds(c * _CHUNK, _CHUNK), :], xbuf.at[slot],
            isem.at[slot])

    def out_copy(c, slot):
        return pltpu.make_async_copy(
            obuf.at[slot], out_hbm.at[pl.ds(c * _HALF, _HALF), :],
            osem.at[slot])

    for s in range(_NBUF):
        in_copy(s, s).start()

    def body(i, _):
        slot = jax.lax.rem(i, _NBUF)
        in_copy(i, slot).wait()

        @pl.when(i >= _NBUF)
        def _():
            out_copy(i - _NBUF, slot).wait()

        xc = xbuf[slot].astype(jnp.bfloat16)
        y = jax.lax.dot_general(
            xc[:_HALF], wl, (((1,), (0,)), ((), ())),
            preferred_element_type=jnp.float32)
        y += jax.lax.dot_general(
            xc[_HALF:], wr, (((1,), (0,)), ((), ())),
            preferred_element_type=jnp.float32)
        obuf[slot] = y.astype(jnp.bfloat16)
        out_copy(i, slot).start()

        nxt = i + _NBUF

        @pl.when(nxt < n_chunks)
        def _():
            in_copy(nxt, slot).start()

        return 0

    jax.lax.fori_loop(0, n_chunks, body, 0)

    for s in range(_NBUF):
        c = n_chunks - _NBUF + s
        out_copy(c, jax.lax.rem(c, _NBUF)).wait()


def kernel(x, W):
    m, d = x.shape
    e = W.shape[0]
    wt = W.T
    zeros = jnp.zeros_like(wt)
    wl = jnp.concatenate([wt, zeros], axis=1)
    wr = jnp.concatenate([zeros, wt], axis=1)
    packed = pl.pallas_call(
        _router_kernel,
        in_specs=[
            pl.BlockSpec(memory_space=pltpu.MemorySpace.HBM),
            pl.BlockSpec(memory_space=pltpu.MemorySpace.VMEM),
            pl.BlockSpec(memory_space=pltpu.MemorySpace.VMEM),
        ],
        out_specs=pl.BlockSpec(memory_space=pltpu.MemorySpace.HBM),
        out_shape=jax.ShapeDtypeStruct((m // 2, 2 * e), jnp.bfloat16),
        scratch_shapes=[
            pltpu.VMEM((_NBUF, _CHUNK, d), jnp.float32),
            pltpu.VMEM((_NBUF, _HALF, 2 * e), jnp.bfloat16),
            pltpu.SemaphoreType.DMA((_NBUF,)),
            pltpu.SemaphoreType.DMA((_NBUF,)),
        ],
    )(x, wl, wr)
    unpacked = packed.reshape(m // _CHUNK, _HALF, 2, e)
    unpacked = unpacked.transpose(0, 2, 1, 3).reshape(m, e)
    return unpacked.astype(jnp.float32)


# speedup vs baseline: 1.2619x; 1.2619x over previous
import jax
import jax.numpy as jnp
from jax.experimental import pallas as pl
from jax.experimental.pallas import tpu as pltpu

_CHUNK = 1024
_NBUF = 10


def _router_kernel(x_hbm, w_ref, out_hbm, xbuf, obuf, isem, osem):
    n_chunks = x_hbm.shape[0] // _CHUNK
    w = w_ref[...].astype(jnp.bfloat16)

    def in_copy(c, slot):
        return pltpu.make_async_copy(
            x_hbm.at[pl.ds(c * _CHUNK, _CHUNK), :], xbuf.at[slot],
            isem.at[slot])

    def out_copy(c, slot):
        return pltpu.make_async_copy(
            obuf.at[slot], out_hbm.at[pl.ds(c * _CHUNK, _CHUNK), :],
            osem.at[slot])

    for s in range(_NBUF):
        in_copy(s, s).start()

    def body(i, _):
        slot = jax.lax.rem(i, _NBUF)
        in_copy(i, slot).wait()

        @pl.when(i >= _NBUF)
        def _():
            out_copy(i - _NBUF, slot).wait()

        xc = xbuf[slot].astype(jnp.bfloat16)
        y = jax.lax.dot_general(
            xc, w, (((1,), (0,)), ((), ())),
            preferred_element_type=jnp.float32)
        obuf[slot] = y.astype(jnp.bfloat16)
        out_copy(i, slot).start()

        nxt = i + _NBUF

        @pl.when(nxt < n_chunks)
        def _():
            in_copy(nxt, slot).start()

        return 0

    jax.lax.fori_loop(0, n_chunks, body, 0)

    for s in range(_NBUF):
        c = n_chunks - _NBUF + s
        out_copy(c, jax.lax.rem(c, _NBUF)).wait()


def kernel(x, W):
    m, d = x.shape
    e = W.shape[0]
    wpad = jnp.concatenate([W.T, jnp.zeros_like(W.T)], axis=1)
    packed = pl.pallas_call(
        _router_kernel,
        in_specs=[
            pl.BlockSpec(memory_space=pltpu.MemorySpace.HBM),
            pl.BlockSpec(memory_space=pltpu.MemorySpace.VMEM),
        ],
        out_specs=pl.BlockSpec(memory_space=pltpu.MemorySpace.HBM),
        out_shape=jax.ShapeDtypeStruct((m, 2 * e), jnp.bfloat16),
        scratch_shapes=[
            pltpu.VMEM((_NBUF, _CHUNK, d), jnp.float32),
            pltpu.VMEM((_NBUF, _CHUNK, 2 * e), jnp.bfloat16),
            pltpu.SemaphoreType.DMA((_NBUF,)),
            pltpu.SemaphoreType.DMA((_NBUF,)),
        ],
    )(x, wpad)
    return packed[:, :e].astype(jnp.float32)


# speedup vs baseline: 1.8404x; 1.4584x over previous
import jax
import jax.numpy as jnp
from jax.experimental import pallas as pl
from jax.experimental.pallas import tpu as pltpu

_CHUNK = 1024
_NBUF = 10


def _router_kernel(x_hbm, w_ref, out_hbm, xbuf, obuf, isem, osem):
    n_chunks = x_hbm.shape[0] // _CHUNK
    w = w_ref[...].astype(jnp.bfloat16)

    def in_copy(c, slot):
        return pltpu.make_async_copy(
            x_hbm.at[pl.ds(c * _CHUNK, _CHUNK), :], xbuf.at[slot],
            isem.at[slot])

    def out_copy(c, slot):
        return pltpu.make_async_copy(
            obuf.at[slot], out_hbm.at[pl.ds(c * _CHUNK, _CHUNK), :],
            osem.at[slot])

    for s in range(_NBUF):
        in_copy(s, s).start()

    def body(i, _):
        slot = jax.lax.rem(i, _NBUF)
        in_copy(i, slot).wait()

        @pl.when(i >= _NBUF)
        def _():
            out_copy(i - _NBUF, slot).wait()

        xc = xbuf[slot].astype(jnp.bfloat16)
        obuf[slot] = jax.lax.dot_general(
            xc, w, (((1,), (1,)), ((), ())),
            preferred_element_type=jnp.float32)
        out_copy(i, slot).start()

        nxt = i + _NBUF

        @pl.when(nxt < n_chunks)
        def _():
            in_copy(nxt, slot).start()

        return 0

    jax.lax.fori_loop(0, n_chunks, body, 0)

    for s in range(_NBUF):
        c = n_chunks - _NBUF + s
        out_copy(c, jax.lax.rem(c, _NBUF)).wait()


def kernel(x, W):
    m, d = x.shape
    e = W.shape[0]
    return pl.pallas_call(
        _router_kernel,
        in_specs=[
            pl.BlockSpec(memory_space=pltpu.MemorySpace.HBM),
            pl.BlockSpec(memory_space=pltpu.MemorySpace.VMEM),
        ],
        out_specs=pl.BlockSpec(memory_space=pltpu.MemorySpace.HBM),
        out_shape=jax.ShapeDtypeStruct((m, e), jnp.float32),
        scratch_shapes=[
            pltpu.VMEM((_NBUF, _CHUNK, d), jnp.float32),
            pltpu.VMEM((_NBUF, _CHUNK, e), jnp.float32),
            pltpu.SemaphoreType.DMA((_NBUF,)),
            pltpu.SemaphoreType.DMA((_NBUF,)),
        ],
    )(x, W)


# speedup vs baseline: 1.8487x; 1.0045x over previous
import jax
import jax.numpy as jnp
from jax.experimental import pallas as pl
from jax.experimental.pallas import tpu as pltpu

_CHUNK = 512
_NBUF = 16


def _router_kernel(x_hbm, w_ref, out_hbm, xbuf, obuf, isem, osem):
    n_chunks = x_hbm.shape[0] // _CHUNK
    w = w_ref[...].astype(jnp.bfloat16)

    def in_copy(c, slot):
        return pltpu.make_async_copy(
            x_hbm.at[pl.ds(c * _CHUNK, _CHUNK), :], xbuf.at[slot],
            isem.at[slot])

    def out_copy(c, slot):
        return pltpu.make_async_copy(
            obuf.at[slot], out_hbm.at[pl.ds(c * _CHUNK, _CHUNK), :],
            osem.at[slot])

    for s in range(_NBUF):
        in_copy(s, s).start()

    def body(i, _):
        slot = jax.lax.rem(i, _NBUF)
        in_copy(i, slot).wait()

        @pl.when(i >= _NBUF)
        def _():
            out_copy(i - _NBUF, slot).wait()

        xc = xbuf[slot].astype(jnp.bfloat16)
        obuf[slot] = jax.lax.dot_general(
            xc, w, (((1,), (1,)), ((), ())),
            preferred_element_type=jnp.float32)
        out_copy(i, slot).start()

        nxt = i + _NBUF

        @pl.when(nxt < n_chunks)
        def _():
            in_copy(nxt, slot).start()

        return 0

    jax.lax.fori_loop(0, n_chunks, body, 0)

    for s in range(_NBUF):
        c = n_chunks - _NBUF + s
        out_copy(c, jax.lax.rem(c, _NBUF)).wait()


def kernel(x, W):
    m, d = x.shape
    e = W.shape[0]
    return pl.pallas_call(
        _router_kernel,
        in_specs=[
            pl.BlockSpec(memory_space=pltpu.MemorySpace.HBM),
            pl.BlockSpec(memory_space=pltpu.MemorySpace.VMEM),
        ],
        out_specs=pl.BlockSpec(memory_space=pltpu.MemorySpace.HBM),
        out_shape=jax.ShapeDtypeStruct((m, e), jnp.float32),
        scratch_shapes=[
            pltpu.VMEM((_NBUF, _CHUNK, d), jnp.float32),
            pltpu.VMEM((_NBUF, _CHUNK, e), jnp.float32),
            pltpu.SemaphoreType.DMA((_NBUF,)),
            pltpu.SemaphoreType.DMA((_NBUF,)),
        ],
    )(x, W)
